# Initial kernel scaffold; baseline (speedup 1.0000x reference)
import functools

import jax
import jax.numpy as jnp
from jax import lax
from jax.experimental import pallas as pl
from jax.experimental.pallas import tpu as pltpu
from jax.experimental.pallas import tpu_sc as plsc

N = 10000
D = 128
NP = 10240
NC = 2
NS = 16
NW = NC * NS
CH = 128
ROWS_PER_SUB = NP // NS
ZR = 64


def _cdiv(a, b):
    return (a + b - 1) // b


def _sc_body(chunks, h_hbm, src_hbm, dst_hbm, out_hbm,
             src_v, dst_v, rows_v, zer_v, agg_sh, gsem):
    c = lax.axis_index("c")
    s = lax.axis_index("s")
    wid = s * NC + c

    zeros16 = jnp.zeros((16,), jnp.float32)

    def zloop(i, carry):
        r = i // (D // 16)
        k = i % (D // 16)
        zer_v[r, pl.ds(k * 16, 16)] = zeros16
        return carry

    lax.fori_loop(0, ZR * (D // 16), zloop, 0)

    row0 = s * ROWS_PER_SUB

    def zcopy(j, carry):
        pltpu.sync_copy(zer_v, agg_sh.at[pl.ds(row0 + j * ZR, ZR)])
        return carry

    lax.fori_loop(0, ROWS_PER_SUB // ZR, zcopy, 0)

    pltpu.sync_copy(src_hbm.at[wid], src_v)
    pltpu.sync_copy(dst_hbm.at[wid], dst_v)

    plsc.subcore_barrier()

    cp = pltpu.async_copy(h_hbm.at[src_v.at[0]], rows_v.at[0], gsem)

    def body(j, carry):
        pltpu.async_copy(h_hbm.at[src_v.at[j + 1]], rows_v.at[(j + 1) % 2], gsem)
        cp.wait()
        pltpu.sync_copy(rows_v.at[j % 2], agg_sh.at[dst_v.at[j]], add=True)
        return carry

    lax.fori_loop(0, chunks - 1, body, 0)
    cp.wait()
    pltpu.sync_copy(rows_v.at[(chunks - 1) % 2],
                    agg_sh.at[dst_v.at[chunks - 1]], add=True)

    plsc.subcore_barrier()

    pltpu.sync_copy(agg_sh.at[pl.ds(row0, ROWS_PER_SUB)],
                    out_hbm.at[c, pl.ds(row0, ROWS_PER_SUB)])


def _make_sc_agg(chunks):
    mesh = plsc.VectorSubcoreMesh(core_axis_name="c", subcore_axis_name="s")
    return pl.kernel(
        functools.partial(_sc_body, chunks),
        out_type=jax.ShapeDtypeStruct((NC, NP, D), jnp.float32),
        mesh=mesh,
        scratch_types=[
            pltpu.VMEM((chunks, CH), jnp.int32),
            pltpu.VMEM((chunks, CH), jnp.int32),
            pltpu.VMEM((2, CH, D), jnp.float32),
            pltpu.VMEM((ZR, D), jnp.float32),
            pltpu.VMEM_SHARED((NP, D), jnp.float32),
            pltpu.SemaphoreType.DMA,
        ],
    )


def _atom_body(x_ref, t_ref, o_ref):
    nf = t_ref.shape[0]
    col = lax.broadcasted_iota(jnp.int32, (1, 128), 1)
    acc = jnp.zeros(o_ref.shape, jnp.float32)
    for i in range(nf):
        oh = (x_ref[:, i:i + 1] == col).astype(jnp.float32)
        acc = acc + jnp.dot(oh, t_ref[i], preferred_element_type=jnp.float32)
    o_ref[...] = acc


def _atom_encode(xp, tables_p, block=512):
    nf = tables_p.shape[0]
    grid = (NP // block,)
    return pl.pallas_call(
        _atom_body,
        grid=grid,
        in_specs=[
            pl.BlockSpec((block, xp.shape[1]), lambda i: (i, 0)),
            pl.BlockSpec((nf, 128, 128), lambda i: (0, 0, 0)),
        ],
        out_specs=pl.BlockSpec((block, D), lambda i: (i, 0)),
        out_shape=jax.ShapeDtypeStruct((NP, D), jnp.float32),
    )(xp, tables_p)


def _combine_body(a_ref, h_ref, wr_ref, wt_ref, b_ref, o_ref):
    agg = a_ref[0] + a_ref[1]
    o_ref[...] = (
        jnp.dot(agg, wr_ref[...], preferred_element_type=jnp.float32)
        + jnp.dot(h_ref[...], wt_ref[...], preferred_element_type=jnp.float32)
        + b_ref[...]
    )


def _combine(parts, h, wr, wt, b, block=512):
    grid = (NP // block,)
    return pl.pallas_call(
        _combine_body,
        grid=grid,
        in_specs=[
            pl.BlockSpec((NC, block, D), lambda i: (0, i, 0)),
            pl.BlockSpec((block, D), lambda i: (i, 0)),
            pl.BlockSpec((D, D), lambda i: (0, 0)),
            pl.BlockSpec((D, D), lambda i: (0, 0)),
            pl.BlockSpec((1, D), lambda i: (0, 0)),
        ],
        out_specs=pl.BlockSpec((block, D), lambda i: (i, 0)),
        out_shape=jax.ShapeDtypeStruct((NP, D), jnp.float32),
    )(parts, h, wr, wt, b)


def kernel(x, edge_index, edge_attr, batch, atom_tables, W_rel, b_rel, W_root):
    del edge_attr, batch
    e = edge_index.shape[1]
    num_layers = W_rel.shape[0]

    xp = jnp.zeros((NP, x.shape[1]), jnp.int32).at[:N].set(x.astype(jnp.int32))
    tables_p = jnp.zeros((atom_tables.shape[0], 128, 128), jnp.float32)
    tables_p = tables_p.at[:, :atom_tables.shape[1], :].set(
        atom_tables.astype(jnp.float32))

    chunks = _cdiv(e, NW * CH)
    ep = NW * chunks * CH
    src = jnp.full((ep,), 0, jnp.int32).at[:e].set(edge_index[0].astype(jnp.int32))
    dst = jnp.full((ep,), N, jnp.int32).at[:e].set(edge_index[1].astype(jnp.int32))
    src = src.reshape(NW, chunks, CH)
    dst = dst.reshape(NW, chunks, CH)

    sc_agg = _make_sc_agg(chunks)

    h = _atom_encode(xp, tables_p)
    for l in range(num_layers):
        parts = sc_agg(h, src, dst)
        h = _combine(parts, h, W_rel[l], W_root[l], b_rel[l].reshape(1, D))
    return h[:N]


# speedup vs baseline: 8.1128x; 8.1128x over previous
import functools

import jax
import jax.numpy as jnp
from jax import lax
from jax.experimental import pallas as pl
from jax.experimental.pallas import tpu as pltpu
from jax.experimental.pallas import tpu_sc as plsc

N = 10000
D = 128
H = D // 2
NP = 10240
NC = 2
NS = 16
CH = 128
ROWS_PER_SUB = NP // NS
ZR = 128


def _cdiv(a, b):
    return (a + b - 1) // b


def _sc_body(chunks, h_hbm, src_hbm, dst_hbm, out_hbm,
             src_v, dst_v, rows_v, zer_v, agg_sh, gsem):
    c = lax.axis_index("c")
    s = lax.axis_index("s")

    zeros16 = jnp.zeros((16,), jnp.float32)

    def zloop(i, carry):
        r = i // (H // 16)
        k = i % (H // 16)
        zer_v[r, pl.ds(k * 16, 16)] = zeros16
        return carry

    lax.fori_loop(0, ZR * (H // 16), zloop, 0)

    row0 = s * ROWS_PER_SUB

    def zcopy(j, carry):
        pltpu.sync_copy(zer_v, agg_sh.at[pl.ds(row0 + j * ZR, ZR)])
        return carry

    lax.fori_loop(0, ROWS_PER_SUB // ZR, zcopy, 0)

    pltpu.sync_copy(src_hbm.at[s], src_v)
    pltpu.sync_copy(dst_hbm.at[s], dst_v)

    plsc.subcore_barrier()

    h_half = h_hbm.at[c]
    cp = pltpu.async_copy(h_half.at[src_v.at[0]], rows_v.at[0], gsem)

    def body(j, carry):
        pltpu.async_copy(h_half.at[src_v.at[j + 1]], rows_v.at[(j + 1) % 2],
                         gsem)
        cp.wait()
        pltpu.sync_copy(rows_v.at[j % 2], agg_sh.at[dst_v.at[j]], add=True)
        return carry

    lax.fori_loop(0, chunks - 1, body, 0)
    cp.wait()
    pltpu.sync_copy(rows_v.at[(chunks - 1) % 2],
                    agg_sh.at[dst_v.at[chunks - 1]], add=True)

    plsc.subcore_barrier()

    pltpu.sync_copy(agg_sh.at[pl.ds(row0, ROWS_PER_SUB)],
                    out_hbm.at[c, pl.ds(row0, ROWS_PER_SUB)])


def _make_sc_agg(chunks):
    mesh = plsc.VectorSubcoreMesh(core_axis_name="c", subcore_axis_name="s")
    return pl.kernel(
        functools.partial(_sc_body, chunks),
        out_type=jax.ShapeDtypeStruct((NC, NP, H), jnp.float32),
        mesh=mesh,
        scratch_types=[
            pltpu.VMEM((chunks, CH), jnp.int32),
            pltpu.VMEM((chunks, CH), jnp.int32),
            pltpu.VMEM((2, CH, H), jnp.float32),
            pltpu.VMEM((ZR, H), jnp.float32),
            pltpu.VMEM_SHARED((NP, H), jnp.float32),
            pltpu.SemaphoreType.DMA,
        ],
        compiler_params=pltpu.CompilerParams(use_tc_tiling_on_sc=False),
    )


def _atom_body(x_ref, t_ref, o_ref):
    nf = t_ref.shape[0]
    col = lax.broadcasted_iota(jnp.int32, (1, 128), 1)
    acc = jnp.zeros((x_ref.shape[0], D), jnp.float32)
    for i in range(nf):
        oh = (x_ref[:, i:i + 1] == col).astype(jnp.float32)
        acc = acc + jnp.dot(oh, t_ref[i], preferred_element_type=jnp.float32)
    o_ref[0] = acc[:, :H]
    o_ref[1] = acc[:, H:]


def _atom_encode(xp, tables_p, block=512):
    nf = tables_p.shape[0]
    grid = (NP // block,)
    return pl.pallas_call(
        _atom_body,
        grid=grid,
        in_specs=[
            pl.BlockSpec((block, xp.shape[1]), lambda i: (i, 0)),
            pl.BlockSpec((nf, 128, 128), lambda i: (0, 0, 0)),
        ],
        out_specs=pl.BlockSpec((NC, block, H), lambda i: (0, i, 0)),
        out_shape=jax.ShapeDtypeStruct((NC, NP, H), jnp.float32),
    )(xp, tables_p)


def _combine_body(a_ref, h_ref, wr_ref, wt_ref, b_ref, o_ref):
    agg = jnp.concatenate([a_ref[0], a_ref[1]], axis=1)
    hb = jnp.concatenate([h_ref[0], h_ref[1]], axis=1)
    out = (
        jnp.dot(agg, wr_ref[...], preferred_element_type=jnp.float32)
        + jnp.dot(hb, wt_ref[...], preferred_element_type=jnp.float32)
        + b_ref[...]
    )
    o_ref[0] = out[:, :H]
    o_ref[1] = out[:, H:]


def _combine(parts, h, wr, wt, b, block=512):
    grid = (NP // block,)
    return pl.pallas_call(
        _combine_body,
        grid=grid,
        in_specs=[
            pl.BlockSpec((NC, block, H), lambda i: (0, i, 0)),
            pl.BlockSpec((NC, block, H), lambda i: (0, i, 0)),
            pl.BlockSpec((D, D), lambda i: (0, 0)),
            pl.BlockSpec((D, D), lambda i: (0, 0)),
            pl.BlockSpec((1, D), lambda i: (0, 0)),
        ],
        out_specs=pl.BlockSpec((NC, block, H), lambda i: (0, i, 0)),
        out_shape=jax.ShapeDtypeStruct((NC, NP, H), jnp.float32),
    )(parts, h, wr, wt, b)


def kernel(x, edge_index, edge_attr, batch, atom_tables, W_rel, b_rel, W_root):
    del edge_attr, batch
    e = edge_index.shape[1]
    num_layers = W_rel.shape[0]

    xp = jnp.zeros((NP, x.shape[1]), jnp.int32).at[:N].set(x.astype(jnp.int32))
    tables_p = jnp.zeros((atom_tables.shape[0], 128, 128), jnp.float32)
    tables_p = tables_p.at[:, :atom_tables.shape[1], :].set(
        atom_tables.astype(jnp.float32))

    chunks = _cdiv(e, NS * CH)
    ep = NS * chunks * CH
    src = jnp.full((ep,), 0, jnp.int32).at[:e].set(edge_index[0].astype(jnp.int32))
    dst = jnp.full((ep,), N, jnp.int32).at[:e].set(edge_index[1].astype(jnp.int32))
    src = src.reshape(NS, chunks, CH)
    dst = dst.reshape(NS, chunks, CH)

    sc_agg = _make_sc_agg(chunks)

    h = _atom_encode(xp, tables_p)
    for l in range(num_layers):
        parts = sc_agg(h, src, dst)
        h = _combine(parts, h, W_rel[l], W_root[l], b_rel[l].reshape(1, D))
    return jnp.concatenate([h[0, :N], h[1, :N]], axis=1)
